# baseline (speedup 1.0000x reference)
import functools

import jax
import jax.numpy as jnp
from jax import lax
from jax.experimental import pallas as pl
from jax.experimental.pallas import tpu as pltpu
from jax.experimental.pallas import tpu_sc as plsc

_info = plsc.get_sparse_core_info()
_NC = _info.num_cores
_NS = _info.num_subcores
_NW = _NC * _NS


def _make_bcast(batch, rows, lanes, dtype):
  b_per_w = batch // _NW
  k = 8
  while b_per_w % k:
    k //= 2
  n_dma = b_per_w // k
  mesh = plsc.VectorSubcoreMesh(core_axis_name="c", subcore_axis_name="s")

  @functools.partial(
      pl.kernel,
      out_type=jax.ShapeDtypeStruct((batch, rows, lanes), dtype),
      mesh=mesh,
      scratch_types=[
          pltpu.VMEM((k, rows, lanes), dtype),
          pltpu.SemaphoreType.DMA,
          pltpu.SemaphoreType.DMA,
      ],
  )
  def bcast(pe_hbm, out_hbm, rep_v, sem_in, sem_out):
    wid = lax.axis_index("s") * _NC + lax.axis_index("c")
    base = wid * b_per_w
    fills = [pltpu.async_copy(pe_hbm, rep_v.at[j], sem_in) for j in range(k)]
    for h in fills:
      h.wait()
    outs = [
        pltpu.async_copy(rep_v, out_hbm.at[pl.ds(base + t * k, k)], sem_out)
        for t in range(n_dma)
    ]
    for h in outs:
      h.wait()

  return bcast


def kernel(x, pe_weight):
  batch = x.shape[0]
  max_len, d_model = pe_weight.shape
  n = max_len * d_model
  if n % 128 == 0:
    rows, lanes = n // 128, 128
  else:
    rows, lanes = max_len, d_model
  pe2 = pe_weight.reshape(rows, lanes)
  out = _make_bcast(batch, rows, lanes, pe_weight.dtype)(pe2)
  return out.reshape(batch, max_len, d_model)
